# baseline (speedup 1.0000x reference)
import jax
import jax.numpy as jnp
from jax import lax
from jax.experimental import pallas as pl
from jax.experimental.pallas import tpu as pltpu
from jax.experimental.pallas import tpu_sc as plsc

NUM_TERMS = 16
NUM_COLS = 16384
DEPTH = 128
NUM_WORKERS = 32
COLS_PER_W = NUM_COLS // NUM_WORKERS
CHUNK = 128
NUM_CHUNKS = COLS_PER_W // CHUNK
LANES = 16
VECS_PER_ROW = DEPTH // LANES


def _body(x_hbm, pe_hbm, out_hbm,
          idx_v, acc0, acc1, acc2, acc3,
          gsem0, gsem1, gsem2, gsem3, wsem0):
    cid = lax.axis_index("c")
    sid = lax.axis_index("s")
    wid = sid * 2 + cid
    col0 = wid * COLS_PER_W
    inv = jnp.float32(1.0 / NUM_TERMS)
    zvec = jnp.zeros((LANES,), jnp.float32)

    accs = (acc0, acc1, acc2, acc3)
    gsems = (gsem0, gsem1, gsem2, gsem3)
    wsems = (wsem0,)

    def zero_acc(acc):
        def zbody(r, carry):
            for j in range(VECS_PER_ROW):
                acc[r, pl.ds(j * LANES, LANES)] = zvec
            return carry
        lax.fori_loop(0, CHUNK, zbody, 0)

    def fire(k):
        return [
            pltpu.async_copy(
                pe_hbm.at[idx_v.at[i, pl.ds(k * CHUNK, CHUNK)]],
                accs[k], gsems[k], add=True)
            for i in range(NUM_TERMS)
        ]

    idx0_cp = pltpu.async_copy(x_hbm.at[:, pl.ds(col0, CHUNK)],
                               idx_v.at[:, pl.ds(0, CHUNK)], wsem0)
    zero_acc(acc0)
    idx0_cp.wait()
    pending = {0: fire(0)}
    idxr_cp = pltpu.async_copy(
        x_hbm.at[:, pl.ds(col0 + CHUNK, COLS_PER_W - CHUNK)],
        idx_v.at[:, pl.ds(CHUNK, COLS_PER_W - CHUNK)], wsem0)
    zero_acc(acc1)
    idxr_cp.wait()
    pending[1] = fire(1)
    for k in range(2, NUM_CHUNKS):
        zero_acc(accs[k])
        pending[k] = fire(k)

    wb = []
    for k in range(NUM_CHUNKS):
        acc = accs[k]
        for cd in pending.pop(k):
            cd.wait()

        def row_body(r2, carry):
            for r in (2 * r2, 2 * r2 + 1):
                for j in range(VECS_PER_ROW):
                    sl = pl.ds(j * LANES, LANES)
                    acc[r, sl] = acc[r, sl] * inv
            return carry

        lax.fori_loop(0, CHUNK // 2, row_body, 0)
        wb.append(pltpu.async_copy(
            acc, out_hbm.at[pl.ds(col0 + k * CHUNK, CHUNK)], wsems[0]))
    for cd in wb:
        cd.wait()


@jax.jit
def kernel(x, position_encoding):
    mesh = plsc.VectorSubcoreMesh(core_axis_name="c", subcore_axis_name="s")
    f = pl.kernel(
        _body,
        mesh=mesh,
        out_type=jax.ShapeDtypeStruct((NUM_COLS, DEPTH), jnp.float32),
        scratch_types=(
            [pltpu.VMEM((NUM_TERMS, COLS_PER_W), jnp.int32)]
            + [pltpu.VMEM((CHUNK, DEPTH), jnp.float32)] * 4
            + [pltpu.SemaphoreType.DMA] * 5
        ),
    )
    return f(x, position_encoding)
